# baseline (speedup 1.0000x reference)
import jax
import jax.numpy as jnp
from jax import lax
from jax.experimental import pallas as pl
from jax.experimental.pallas import tpu as pltpu
from jax.experimental.pallas import tpu_sc as plsc

NUM_EMB = 1000000
DIM = 64
BATCH = 16384

NUM_CORES = 2
NUM_SUBCORES = 16
NUM_WORKERS = NUM_CORES * NUM_SUBCORES
B_PER_W = BATCH // NUM_WORKERS
NBUF = 8
HALF = B_PER_W // 2
GROUPS = HALF // NBUF


def _fire(tabT_hbm, blocks, sems, s, xq):
    q = lax.max(lax.min(lax.shift_right_logical(xq[s], 7),
                        (NUM_EMB - 1) >> 7), 0)
    col = pl.multiple_of(lax.shift_left(q, 7), 128)
    pltpu.async_copy(tabT_hbm.at[:, pl.ds(col, 128)], blocks.at[s], sems[s])


def _body(x_hbm, tabT_hbm, out_hbm, xv, blocks, rowstage,
          s0, s1, s2, s3, s4, s5, s6, s7):
    sems = (s0, s1, s2, s3, s4, s5, s6, s7)
    wid = lax.axis_index("s") * NUM_CORES + lax.axis_index("c")
    base = wid * B_PER_W

    pltpu.sync_copy(x_hbm.at[pl.ds(base, B_PER_W)], xv.at[pl.ds(0, B_PER_W)])

    xq0 = xv[pl.ds(0, 16)]
    for s in range(NBUF):
        _fire(tabT_hbm, blocks, sems, s, xq0)

    lane = lax.iota(jnp.int32, 16)

    def make_group(half):
        def group(g, _):
            hbase = half * HALF + g * NBUF
            xq = xv[pl.ds(hbase, 16)]
            xqn = xv[pl.ds(hbase + NBUF, 16)]
            for s in range(NBUF):
                m = xq[s] & 127
                mv = lax.broadcast(m, (16,))
                pltpu.make_async_copy(
                    tabT_hbm.at[:, pl.ds(0, 128)], blocks.at[s],
                    sems[s]).wait()
                for k in range(DIM // 16):
                    vals = plsc.load_gather(
                        blocks, [lax.broadcast(s, (16,)), lane + k * 16, mv])
                    rowstage[g * NBUF + s, pl.ds(k * 16, 16)] = vals
                _fire(tabT_hbm, blocks, sems, s, xqn)
            return _
        return group

    for half in range(2):
        lax.fori_loop(0, GROUPS, make_group(half), None)
        pltpu.sync_copy(
            rowstage, out_hbm.at[pl.ds(base + half * HALF, HALF)])

    for s in range(NBUF):
        pltpu.make_async_copy(
            tabT_hbm.at[:, pl.ds(0, 128)], blocks.at[s], sems[s]).wait()


@jax.jit
def kernel(x, table):
    xi = x.astype(jnp.int32)
    tab_t = table.T
    mesh = plsc.VectorSubcoreMesh(
        core_axis_name="c", subcore_axis_name="s",
        num_cores=NUM_CORES, num_subcores=NUM_SUBCORES)
    run = pl.kernel(
        _body,
        out_type=jax.ShapeDtypeStruct((BATCH, 2 * DIM), jnp.float32),
        mesh=mesh,
        scratch_types=[
            pltpu.VMEM((B_PER_W + 16,), jnp.int32),
            pltpu.VMEM((NBUF, DIM, 2 * DIM), jnp.float32),
            pltpu.VMEM((HALF, 2 * DIM), jnp.float32),
        ] + [pltpu.SemaphoreType.DMA] * NBUF,
        compiler_params=pltpu.CompilerParams(
            needs_layout_passes=False, disable_bounds_checks=True),
    )
    xs, order = lax.sort_key_val(xi, jnp.arange(BATCH, dtype=jnp.int32))
    inv = jnp.zeros((BATCH,), jnp.int32).at[order].set(
        jnp.arange(BATCH, dtype=jnp.int32))
    out2 = run(xs, tab_t)
    return jnp.take(out2, inv, axis=0)[:, :DIM]


# speedup vs baseline: 1.7590x; 1.7590x over previous
import jax
import jax.numpy as jnp
from jax import lax
from jax.experimental import pallas as pl
from jax.experimental.pallas import tpu as pltpu
from jax.experimental.pallas import tpu_sc as plsc

NUM_EMB = 1000000
DIM = 64
BATCH = 16384

NUM_CORES = 2
NUM_SUBCORES = 16
NUM_WORKERS = NUM_CORES * NUM_SUBCORES
B_PER_W = BATCH // NUM_WORKERS
NBUF = 6
LPAD = 544
Q_MAX = (NUM_EMB - 1) >> 7


def _fire(tabT_hbm, blocks, sems, s, qval):
    q = lax.max(lax.min(qval, Q_MAX), 0)
    col = pl.multiple_of(lax.shift_left(q, 7), 128)
    pltpu.async_copy(tabT_hbm.at[:, pl.ds(col, 128)], blocks.at[s], sems[s])


def _body(x_hbm, tabT_hbm, out_hbm, xv, qlist, runstart, blocks, rowstage,
          s0, s1, s2, s3, s4, s5):
    sems = (s0, s1, s2, s3, s4, s5)
    wid = lax.axis_index("s") * NUM_CORES + lax.axis_index("c")
    base = wid * B_PER_W

    pltpu.sync_copy(x_hbm.at[pl.ds(base, B_PER_W)], xv.at[pl.ds(0, B_PER_W)])

    lane = lax.iota(jnp.int32, 16)
    endv = lax.broadcast(jnp.int32(B_PER_W), (16,))

    def fill(v, _):
        runstart[pl.ds(v * 16, 16)] = endv
        return _

    lax.fori_loop(0, LPAD // 16, fill, None)

    def scan(g, off):
        il = lane + g * 16
        qv = lax.shift_right_logical(xv[pl.ds(g * 16, 16)], 7)
        xprev = plsc.load_gather(xv, [lax.max(il - 1, 0)])
        qprev = lax.shift_right_logical(xprev, 7)
        isnew = (qv != qprev) | (il == 0)
        plsc.store_compressed(qlist.at[pl.ds(off, 16)], qv, mask=isnew)
        plsc.store_compressed(runstart.at[pl.ds(off, 16)], il, mask=isnew)
        return off + plsc.all_reduce_population_count(isnew)[0]

    nruns = lax.fori_loop(0, B_PER_W // 16, scan, jnp.int32(0))
    rounds = lax.div(nruns + (NBUF - 1), jnp.int32(NBUF))

    q0 = qlist[pl.ds(0, 16)]
    for s in range(NBUF):
        _fire(tabT_hbm, blocks, sems, s, q0[s])

    def round_body(r, _):
        rs = runstart[pl.ds(r * NBUF, 16)]
        qn = qlist[pl.ds((r + 1) * NBUF, 16)]
        for s in range(NBUF):
            pltpu.make_async_copy(
                tabT_hbm.at[:, pl.ds(0, 128)], blocks.at[s], sems[s]).wait()

            def hit(h, _h):
                xh = xv[pl.ds(h, 16)][0]
                mv = lax.broadcast(xh & 127, (16,))
                sv = lax.broadcast(jnp.int32(s), (16,))
                for k in range(DIM // 16):
                    vals = plsc.load_gather(blocks, [sv, lane + k * 16, mv])
                    rowstage[h, pl.ds(k * 16, 16)] = vals
                return _h

            lax.fori_loop(rs[s], rs[s + 1], hit, None)
            _fire(tabT_hbm, blocks, sems, s, qn[s])
        return _

    lax.fori_loop(0, rounds, round_body, None)

    for s in range(NBUF):
        pltpu.make_async_copy(
            tabT_hbm.at[:, pl.ds(0, 128)], blocks.at[s], sems[s]).wait()

    pltpu.sync_copy(rowstage, out_hbm.at[pl.ds(base, B_PER_W)])


@jax.jit
def kernel(x, table):
    xi = x.astype(jnp.int32)
    tab_t = table.T
    mesh = plsc.VectorSubcoreMesh(
        core_axis_name="c", subcore_axis_name="s",
        num_cores=NUM_CORES, num_subcores=NUM_SUBCORES)
    run = pl.kernel(
        _body,
        out_type=jax.ShapeDtypeStruct((BATCH, 2 * DIM), jnp.float32),
        mesh=mesh,
        scratch_types=[
            pltpu.VMEM((B_PER_W + 16,), jnp.int32),
            pltpu.VMEM((LPAD,), jnp.int32),
            pltpu.VMEM((LPAD,), jnp.int32),
            pltpu.VMEM((NBUF, DIM, 2 * DIM), jnp.float32),
            pltpu.VMEM((B_PER_W, 2 * DIM), jnp.float32),
        ] + [pltpu.SemaphoreType.DMA] * NBUF,
        compiler_params=pltpu.CompilerParams(
            needs_layout_passes=False, disable_bounds_checks=True),
    )
    xs, order = lax.sort_key_val(xi, jnp.arange(BATCH, dtype=jnp.int32))
    inv = jnp.zeros((BATCH,), jnp.int32).at[order].set(
        jnp.arange(BATCH, dtype=jnp.int32))
    out2 = run(xs, tab_t)
    return jnp.take(out2, inv, axis=0)[:, :DIM]
